# baseline (speedup 1.0000x reference)
import jax
import jax.numpy as jnp
from jax import lax
from jax.experimental import pallas as pl
from jax.experimental.pallas import tpu as pltpu
from jax.experimental.pallas import tpu_sc as plsc

_VOCAB = 28
_N_ALPHA = 20
_ALPHA_OFFSET = 3
_BLOSUM_DIM = 24
_ROW_PAD = 128

_NC = 2
_NS = 16
_NW = _NC * _NS
_LANES = 16
_GCHUNK = 128
_SUB = 512


def _sc_gather_body(src_hbm, table_hbm, out_hbm, idx_v, rows_v, sem):
    ntok = idx_v.shape[0]
    wid = lax.axis_index("s") * _NC + lax.axis_index("c")
    base = wid * ntok
    pltpu.sync_copy(src_hbm.at[pl.ds(base, ntok)], idx_v)

    def clamp(i, carry):
        v = idx_v[pl.ds(i * _LANES, _LANES)]
        valid = (v >= _ALPHA_OFFSET) & (v < _ALPHA_OFFSET + _N_ALPHA)
        idx_v[pl.ds(i * _LANES, _LANES)] = jnp.where(valid, v, _VOCAB - 1)
        return carry

    lax.fori_loop(0, ntok // _LANES, clamp, 0)

    for sub in range(ntok // _SUB):
        copies = [
            pltpu.async_copy(
                table_hbm.at[idx_v.at[pl.ds(sub * _SUB + j * _GCHUNK, _GCHUNK)]],
                rows_v.at[pl.ds(j * _GCHUNK, _GCHUNK)],
                sem,
            )
            for j in range(_SUB // _GCHUNK)
        ]
        for c in copies:
            c.wait()
        pltpu.sync_copy(rows_v, out_hbm.at[pl.ds(base + sub * _SUB, _SUB), :])


def _sc_gather(srcf, table):
    n = srcf.shape[0]
    ntok = n // _NW
    mesh = plsc.VectorSubcoreMesh(core_axis_name="c", subcore_axis_name="s")
    f = pl.kernel(
        _sc_gather_body,
        out_type=jax.ShapeDtypeStruct((n, _ROW_PAD), jnp.float32),
        mesh=mesh,
        scratch_types=[
            pltpu.VMEM((ntok,), jnp.int32),
            pltpu.VMEM((_SUB, _ROW_PAD), jnp.float32),
            pltpu.SemaphoreType.DMA,
        ],
    )
    return f(srcf, table)


def _tc_concat_body(x_ref, cod_ref, out_ref):
    out_ref[0] = jnp.concatenate(
        [x_ref[0], cod_ref[0][:, :_BLOSUM_DIM]], axis=1
    )


def kernel(src, x, blosum):
    B, L, D = x.shape
    table = jnp.pad(blosum, ((0, 0), (0, _ROW_PAD - _BLOSUM_DIM)))
    srcf = src.astype(jnp.int32).reshape(B * L)
    coding = _sc_gather(srcf, table).reshape(B, L, _ROW_PAD)
    out = pl.pallas_call(
        _tc_concat_body,
        grid=(B,),
        in_specs=[
            pl.BlockSpec((1, L, D), lambda b: (b, 0, 0)),
            pl.BlockSpec((1, L, _ROW_PAD), lambda b: (b, 0, 0)),
        ],
        out_specs=pl.BlockSpec((1, L, D + _BLOSUM_DIM), lambda b: (b, 0, 0)),
        out_shape=jax.ShapeDtypeStruct((B, L, D + _BLOSUM_DIM), jnp.float32),
    )(x, coding)
    return out


# speedup vs baseline: 3.6140x; 3.6140x over previous
import jax
import jax.numpy as jnp
from jax import lax
from jax.experimental import pallas as pl
from jax.experimental.pallas import tpu as pltpu
from jax.experimental.pallas import tpu_sc as plsc

_VOCAB = 28
_N_ALPHA = 20
_ALPHA_OFFSET = 3
_BLOSUM_DIM = 24
_ROW_PAD = 128

_NC = 2
_NS = 16
_NW = _NC * _NS
_LANES = 16
_GCHUNK = 128
_SUB = 512


def _sc_gather_body(src_hbm, table_hbm, out_hbm, idx_v, table_v, rows_v, sem):
    ntok = idx_v.shape[0]
    wid = lax.axis_index("s") * _NC + lax.axis_index("c")
    base = wid * ntok
    @pl.when(lax.axis_index("s") == 0)
    def _stage_table():
        pltpu.sync_copy(table_hbm, table_v)

    pltpu.sync_copy(src_hbm.at[pl.ds(base, ntok)], idx_v)
    plsc.subcore_barrier()

    def clamp(i, carry):
        v = idx_v[pl.ds(i * _LANES, _LANES)]
        valid = (v >= _ALPHA_OFFSET) & (v < _ALPHA_OFFSET + _N_ALPHA)
        idx_v[pl.ds(i * _LANES, _LANES)] = jnp.where(valid, v, _VOCAB - 1)
        return carry

    lax.fori_loop(0, ntok // _LANES, clamp, 0)

    for sub in range(ntok // _SUB):
        copies = [
            pltpu.async_copy(
                table_v.at[idx_v.at[pl.ds(sub * _SUB + j * _GCHUNK, _GCHUNK)]],
                rows_v.at[pl.ds(j * _GCHUNK, _GCHUNK)],
                sem,
            )
            for j in range(_SUB // _GCHUNK)
        ]
        for c in copies:
            c.wait()
        pltpu.sync_copy(rows_v, out_hbm.at[pl.ds(base + sub * _SUB, _SUB), :])


def _sc_gather(srcf, table):
    n = srcf.shape[0]
    ntok = n // _NW
    mesh = plsc.VectorSubcoreMesh(core_axis_name="c", subcore_axis_name="s")
    f = pl.kernel(
        _sc_gather_body,
        out_type=jax.ShapeDtypeStruct((n, _ROW_PAD), jnp.float32),
        mesh=mesh,
        scratch_types=[
            pltpu.VMEM((ntok,), jnp.int32),
            pltpu.VMEM_SHARED((_VOCAB, _ROW_PAD), jnp.float32),
            pltpu.VMEM((_SUB, _ROW_PAD), jnp.float32),
            pltpu.SemaphoreType.DMA,
        ],
    )
    return f(srcf, table)


def _tc_concat_body(x_ref, cod_ref, out_ref):
    out_ref[0] = jnp.concatenate(
        [x_ref[0], cod_ref[0][:, :_BLOSUM_DIM]], axis=1
    )


def kernel(src, x, blosum):
    B, L, D = x.shape
    table = jnp.pad(blosum, ((0, 0), (0, _ROW_PAD - _BLOSUM_DIM)))
    srcf = src.astype(jnp.int32).reshape(B * L)
    coding = _sc_gather(srcf, table).reshape(B, L, _ROW_PAD)
    out = pl.pallas_call(
        _tc_concat_body,
        grid=(B,),
        in_specs=[
            pl.BlockSpec((1, L, D), lambda b: (b, 0, 0)),
            pl.BlockSpec((1, L, _ROW_PAD), lambda b: (b, 0, 0)),
        ],
        out_specs=pl.BlockSpec((1, L, D + _BLOSUM_DIM), lambda b: (b, 0, 0)),
        out_shape=jax.ShapeDtypeStruct((B, L, D + _BLOSUM_DIM), jnp.float32),
    )(x, coding)
    return out
